# baseline (speedup 1.0000x reference)
import jax
import jax.numpy as jnp
from jax import lax
from jax.experimental import pallas as pl
from jax.experimental.pallas import tpu as pltpu
from jax.experimental.pallas import tpu_sc as plsc

NUSER = 1000000
DIM = 64
B_TOTAL = 4096
L_TOTAL = 200
LANES = 16
NUM_CORES = 2
NUM_SUBCORES = 16
NW = NUM_CORES * NUM_SUBCORES
CHUNK = 256


def _sc_kernel(ids_hbm, tg_hbm, emb_hbm, tab_hbm, out_hbm,
               idx0, idx1, tgv0, tgv1, rw0, rw1, em0, em1,
               sg0, sg1, sl0, sl1, so0, so1, si0, si1):
    idx = (idx0, idx1)
    tgv = (tgv0, tgv1)
    rw = (rw0, rw1)
    em = (em0, em1)
    sg = (sg0, sg1)
    sl = (sl0, sl1)
    so = (so0, so1)
    si = (si0, si1)
    c = lax.axis_index("c")
    s = lax.axis_index("s")
    wid = s * NUM_CORES + c
    n_total = L_TOTAL * B_TOTAL
    n_per_w = n_total // NW
    nchunks = n_per_w // CHUNK
    n0 = wid * n_per_w

    iota = lax.iota(jnp.int32, LANES)

    def emb_slice(ci):
        gn = n0 + ci * CHUNK
        l = gn // B_TOTAL
        b_hi0 = (gn - l * B_TOTAL) // 128
        return l, b_hi0

    def issue_ids(ci, b):
        gn = n0 + ci * CHUNK
        pltpu.async_copy(ids_hbm.at[pl.ds(gn, CHUNK)], idx[b], si[b])

    def wait_ids(ci, b):
        gn = n0 + ci * CHUNK
        pltpu.make_async_copy(ids_hbm.at[pl.ds(gn, CHUNK)], idx[b],
                              si[b]).wait()

    GSPLIT = 4

    def issue_rest(ci, b):
        gn = n0 + ci * CHUNK
        l, b_hi0 = emb_slice(ci)
        for g in range(GSPLIT):
            w = CHUNK // GSPLIT
            pltpu.async_copy(
                tab_hbm.at[idx[b].at[pl.ds(g * w, w)]],
                rw[b].at[pl.ds(g * w, w)], sg[b])
        pltpu.async_copy(tg_hbm.at[pl.ds(gn, CHUNK)], tgv[b], sl[b])
        pltpu.async_copy(
            emb_hbm.at[l, pl.ds(0, DIM // 8), pl.ds(b_hi0, CHUNK // 128)],
            em[b], sl[b])

    def wait_in(ci, b):
        gn = n0 + ci * CHUNK
        l, b_hi0 = emb_slice(ci)
        for g in range(GSPLIT):
            w = CHUNK // GSPLIT
            pltpu.make_async_copy(
                tab_hbm.at[idx[b].at[pl.ds(g * w, w)]],
                rw[b].at[pl.ds(g * w, w)], sg[b]).wait()
        pltpu.make_async_copy(tg_hbm.at[pl.ds(gn, CHUNK)], tgv[b], sl[b]).wait()
        pltpu.make_async_copy(
            emb_hbm.at[l, pl.ds(0, DIM // 8), pl.ds(b_hi0, CHUNK // 128)],
            em[b], sl[b]).wait()

    def wait_out(ci, b):
        l, b_hi0 = emb_slice(ci)
        pltpu.make_async_copy(
            em[b],
            out_hbm.at[l, pl.ds(0, DIM // 8), pl.ds(b_hi0, CHUNK // 128)],
            so[b]).wait()

    def compute(b):
        @plsc.parallel_loop(0, CHUNK // LANES)
        def pos_body(j):
            p = j * LANES
            t = tgv[b][pl.ds(p, LANES)]
            row_idx = p + iota
            blk = j // 8
            off = (j - blk * 8) * LANES
            for grp in range(4):
                feats = range(grp * 16, grp * 16 + 16)
                shs = [plsc.load_gather(
                    rw[b], [row_idx, jnp.full((LANES,), d, jnp.int32)])
                       for d in feats]
                es = [em[b][d // 8, blk, pl.ds((d % 8) * 128 + off, LANES)]
                      for d in feats]
                sigs = [1.0 / (1.0 + jnp.exp(-(t * sh))) for sh in shs]
                for i, d in enumerate(feats):
                    em[b][d // 8, blk, pl.ds((d % 8) * 128 + off, LANES)] = (
                        es[i] * (1.0 + sigs[i]))

    issue_ids(0, 0)
    wait_ids(0, 0)
    issue_rest(0, 0)
    issue_ids(1, 1)

    def pair_body(i2, carry):
        for b in range(2):
            ci = i2 * 2 + b
            nb = 1 - b

            @pl.when(ci >= 1)
            def _():
                wait_out(ci - 1, nb)

            @pl.when(ci + 1 < nchunks)
            def _():
                wait_ids(ci + 1, nb)
                issue_rest(ci + 1, nb)

            wait_in(ci, b)

            @pl.when(ci + 2 < nchunks)
            def _():
                issue_ids(ci + 2, b)

            compute(b)
            l, b_hi0 = emb_slice(ci)
            pltpu.async_copy(
                em[b],
                out_hbm.at[l, pl.ds(0, DIM // 8), pl.ds(b_hi0, CHUNK // 128)],
                so[b])
        return carry

    lax.fori_loop(0, nchunks // 2, pair_body, 0)
    wait_out(nchunks - 1, (nchunks - 1) % 2)


@jax.jit
def _dspp_sc(ids_t, tg_t, emb_4d, table):
    mesh = plsc.VectorSubcoreMesh(core_axis_name="c", subcore_axis_name="s")
    run = pl.kernel(
        _sc_kernel,
        out_type=jax.ShapeDtypeStruct(
            (L_TOTAL, DIM // 8, B_TOTAL // 128, 1024), jnp.float32),
        mesh=mesh,
        scratch_types=[
            pltpu.VMEM((CHUNK,), jnp.int32),
            pltpu.VMEM((CHUNK,), jnp.int32),
            pltpu.VMEM((CHUNK,), jnp.float32),
            pltpu.VMEM((CHUNK,), jnp.float32),
            pltpu.VMEM((CHUNK, DIM), jnp.float32),
            pltpu.VMEM((CHUNK, DIM), jnp.float32),
            pltpu.VMEM((DIM // 8, CHUNK // 128, 1024), jnp.float32),
            pltpu.VMEM((DIM // 8, CHUNK // 128, 1024), jnp.float32),
            pltpu.SemaphoreType.DMA,
            pltpu.SemaphoreType.DMA,
            pltpu.SemaphoreType.DMA,
            pltpu.SemaphoreType.DMA,
            pltpu.SemaphoreType.DMA,
            pltpu.SemaphoreType.DMA,
            pltpu.SemaphoreType.DMA,
            pltpu.SemaphoreType.DMA,
        ],
        compiler_params=pltpu.CompilerParams(
            use_tc_tiling_on_sc=False, needs_layout_passes=False),
    )
    return run(ids_t, tg_t, emb_4d, table)


def kernel(id, emb, time_gap, user_shift_table):
    B, L = id.shape
    n = B * L
    emb_4d = (emb.transpose(1, 2, 0)
              .reshape(L, DIM // 8, 8, B // 128, 128)
              .transpose(0, 1, 3, 2, 4)
              .reshape(L, DIM // 8, B // 128, 1024))
    out_4d = _dspp_sc(
        id.T.reshape(n).astype(jnp.int32),
        time_gap.T.reshape(n),
        emb_4d,
        user_shift_table,
    )
    out = (out_4d.reshape(L, DIM // 8, B // 128, 8, 128)
           .transpose(0, 1, 3, 2, 4)
           .reshape(L, DIM, B)
           .transpose(2, 0, 1))
    return out
